# Initial kernel scaffold; baseline (speedup 1.0000x reference)
import jax
import jax.numpy as jnp
from jax.experimental import pallas as pl


def kernel(src_ids, tgt_ids, src_table, tgt_table):
    raise NotImplementedError("write your pallas kernel here")


# speedup vs baseline: 4.9718x; 4.9718x over previous
import functools

import jax
import jax.numpy as jnp
from jax import lax
from jax.experimental import pallas as pl
from jax.experimental.pallas import tpu as pltpu
from jax.experimental.pallas import tpu_sc as plsc

B, L, D = 4096, 50, 64
N = B * L
C = 128
NC, NS = 2, 16
NW = NC * NS
CPW = N // (NW * C)
G = 5
NSC = CPW // G
SCROWS = G * C


def _sc_body(src_ids, tgt_ids, src_table, tgt_table, src_out, tgt_out,
             idx_s, idx_t, buf0, buf1, gsem0, gsem1, ssem0, ssem1):
    cid = lax.axis_index("c")
    sid = lax.axis_index("s")
    wid = sid * NC + cid
    chunk_base = wid * CPW
    row_base = chunk_base * C

    pltpu.sync_copy(src_ids.at[wid], idx_s)
    pltpu.sync_copy(tgt_ids.at[wid], idx_t)

    bufs = (buf0, buf1)
    gsems = (gsem0, gsem1)
    ssems = (ssem0, ssem1)

    def process_table(table, idx, out):
        gh = {}
        sh = {}

        def start_gathers(j):
            b = j % 2
            hs = []
            for g in range(G):
                ch = j * G + g
                hs.append(pltpu.async_copy(
                    table.at[idx.at[ch]],
                    bufs[b].at[pl.ds(g * C, C)],
                    gsems[b]))
            gh[j] = hs

        start_gathers(0)
        for j in range(NSC):
            b = j % 2
            if j + 1 < NSC:
                if j - 1 >= 0:
                    sh.pop(j - 1).wait()
                start_gathers(j + 1)
            for h in gh.pop(j):
                h.wait()
            sh[j] = pltpu.async_copy(
                bufs[b], out.at[pl.ds(row_base + j * SCROWS, SCROWS)],
                ssems[b])
        for j in sorted(sh):
            sh.pop(j).wait()

    process_table(src_table, idx_s, src_out)
    process_table(tgt_table, idx_t, tgt_out)


@functools.partial(
    pl.kernel,
    out_type=(jax.ShapeDtypeStruct((N, D), jnp.float32),
              jax.ShapeDtypeStruct((N, D), jnp.float32)),
    mesh=plsc.VectorSubcoreMesh(core_axis_name="c", subcore_axis_name="s"),
    compiler_params=pltpu.CompilerParams(use_tc_tiling_on_sc=False),
    scratch_types=[
        pltpu.VMEM((CPW, C), jnp.int32),
        pltpu.VMEM((CPW, C), jnp.int32),
        pltpu.VMEM((SCROWS, D), jnp.float32),
        pltpu.VMEM((SCROWS, D), jnp.float32),
        pltpu.SemaphoreType.DMA,
        pltpu.SemaphoreType.DMA,
        pltpu.SemaphoreType.DMA,
        pltpu.SemaphoreType.DMA,
    ],
)
def _gather_kernel(src_ids, tgt_ids, src_table, tgt_table, src_out, tgt_out,
                   idx_s, idx_t, buf0, buf1, gsem0, gsem1, ssem0, ssem1):
    _sc_body(src_ids, tgt_ids, src_table, tgt_table, src_out, tgt_out,
             idx_s, idx_t, buf0, buf1, gsem0, gsem1, ssem0, ssem1)


def kernel(src_ids, tgt_ids, src_table, tgt_table):
    s_ids = src_ids.reshape(NW, CPW, C).astype(jnp.int32)
    t_ids = tgt_ids.reshape(NW, CPW, C).astype(jnp.int32)
    src_e, tgt_e = _gather_kernel(s_ids, t_ids, src_table, tgt_table)
    return src_e.reshape(B, L, D), tgt_e.reshape(B, L, D)


# speedup vs baseline: 6.3646x; 1.2801x over previous
import functools

import jax
import jax.numpy as jnp
from jax import lax
from jax.experimental import pallas as pl
from jax.experimental.pallas import tpu as pltpu
from jax.experimental.pallas import tpu_sc as plsc

B, L, D = 4096, 50, 64
V = 100000
N = B * L
NC, NS = 2, 16
RPW = D // NS
HALF_L = L // 2


def _gather_chunk(row_v, idx_v, out_v):
    def body(i2, c):
        for u in range(8):
            idx = idx_v[pl.ds(i2 * 128 + u * 16, 16)]
            vals = plsc.load_gather(row_v, [idx])
            out_v[i2, pl.ds(u * 16, 16)] = vals
        return c

    lax.fori_loop(0, B // 128, body, 0)


def _sc_body(ids_s, ids_t, tab_s, tab_t, out_s, out_t,
             row_v, idx_a, idx_b, out_a, out_b,
             isem_a, isem_b, osem_a, osem_b):
    cid = lax.axis_index("c")
    sid = lax.axis_index("s")

    def do_table(ids, tab, out):
        d0 = sid * RPW
        for j in range(RPW):
            d = d0 + j
            kd = d // 8
            sub = d % 8
            pltpu.sync_copy(tab.at[d], row_v)
            pltpu.async_copy(ids.at[pl.ds(0, B)], idx_a, isem_a)

            def li(i, c):
                l0 = 2 * i
                l1 = 2 * i + 1
                pltpu.make_async_copy(ids.at[pl.ds(l0 * B, B)],
                                      idx_a, isem_a).wait()
                pltpu.async_copy(ids.at[pl.ds(l1 * B, B)], idx_b, isem_b)

                @pl.when(i > 0)
                def _():
                    pltpu.make_async_copy(
                        out_a, out.at[l0 - 2, kd, :, sub], osem_a).wait()

                _gather_chunk(row_v, idx_a, out_a)
                pltpu.async_copy(out_a, out.at[l0, kd, :, sub], osem_a)
                pltpu.make_async_copy(ids.at[pl.ds(l1 * B, B)],
                                      idx_b, isem_b).wait()

                @pl.when(i < HALF_L - 1)
                def _():
                    pltpu.async_copy(ids.at[pl.ds((l0 + 2) * B, B)],
                                     idx_a, isem_a)

                @pl.when(i > 0)
                def _():
                    pltpu.make_async_copy(
                        out_b, out.at[l1 - 2, kd, :, sub], osem_b).wait()

                _gather_chunk(row_v, idx_b, out_b)
                pltpu.async_copy(out_b, out.at[l1, kd, :, sub], osem_b)
                return c

            lax.fori_loop(0, HALF_L, li, 0)
            pltpu.make_async_copy(out_a, out.at[L - 2, kd, :, sub],
                                  osem_a).wait()
            pltpu.make_async_copy(out_b, out.at[L - 1, kd, :, sub],
                                  osem_b).wait()

    @pl.when(cid == 0)
    def _():
        do_table(ids_s, tab_s, out_s)

    @pl.when(cid == 1)
    def _():
        do_table(ids_t, tab_t, out_t)


_OUT5 = jax.ShapeDtypeStruct((L, D // 8, B // 128, 8, 128), jnp.float32)


@functools.partial(
    pl.kernel,
    out_type=(_OUT5, _OUT5),
    mesh=plsc.VectorSubcoreMesh(core_axis_name="c", subcore_axis_name="s"),
    compiler_params=pltpu.CompilerParams(
        use_tc_tiling_on_sc=False,
        needs_layout_passes=False,
    ),
    scratch_types=[
        pltpu.VMEM((V,), jnp.float32),
        pltpu.VMEM((B,), jnp.int32),
        pltpu.VMEM((B,), jnp.int32),
        pltpu.VMEM((B // 128, 128), jnp.float32),
        pltpu.VMEM((B // 128, 128), jnp.float32),
        pltpu.SemaphoreType.DMA,
        pltpu.SemaphoreType.DMA,
        pltpu.SemaphoreType.DMA,
        pltpu.SemaphoreType.DMA,
    ],
)
def _emb_kernel(ids_s, ids_t, tab_s, tab_t, out_s, out_t,
                row_v, idx_a, idx_b, out_a, out_b,
                isem_a, isem_b, osem_a, osem_b):
    _sc_body(ids_s, ids_t, tab_s, tab_t, out_s, out_t,
             row_v, idx_a, idx_b, out_a, out_b,
             isem_a, isem_b, osem_a, osem_b)


def kernel(src_ids, tgt_ids, src_table, tgt_table):
    ids_s = src_ids.T.reshape(-1).astype(jnp.int32)
    ids_t = tgt_ids.T.reshape(-1).astype(jnp.int32)
    o_s, o_t = _emb_kernel(ids_s, ids_t, src_table.T, tgt_table.T)

    def finish(o):
        return (o.transpose(0, 1, 3, 2, 4)
                 .reshape(L, D, B)
                 .transpose(2, 0, 1))

    return finish(o_s), finish(o_t)


# speedup vs baseline: 7.0995x; 1.1155x over previous
import functools

import jax
import jax.numpy as jnp
from jax import lax
from jax.experimental import pallas as pl
from jax.experimental.pallas import tpu as pltpu
from jax.experimental.pallas import tpu_sc as plsc

B, L, D = 4096, 50, 64
V = 100000
N = B * L
NC, NS = 2, 16
RPW = D // NS
HALF_L = L // 2


def _gather_chunk(row_v, idx_v, out_v):
    @plsc.parallel_loop(0, B // 16, 1, unroll=8)
    def _(b1):
        idx = idx_v[pl.ds(b1 * 16, 16)]
        vals = plsc.load_gather(row_v, [idx])
        out_v[b1 // 8, pl.ds((b1 % 8) * 16, 16)] = vals


def _sc_body(ids_s, ids_t, tab_s, tab_t, out_s, out_t,
             row_v, idx_a, idx_b, out_a, out_b,
             isem_a, isem_b, osem_a, osem_b):
    cid = lax.axis_index("c")
    sid = lax.axis_index("s")

    def do_table(ids, tab, out):
        d0 = sid * RPW
        for j in range(RPW):
            d = d0 + j
            kd = d // 8
            sub = d % 8
            pltpu.sync_copy(tab.at[d], row_v)
            pltpu.async_copy(ids.at[pl.ds(0, B)], idx_a, isem_a)

            def li(i, c):
                l0 = 2 * i
                l1 = 2 * i + 1
                pltpu.make_async_copy(ids.at[pl.ds(l0 * B, B)],
                                      idx_a, isem_a).wait()
                pltpu.async_copy(ids.at[pl.ds(l1 * B, B)], idx_b, isem_b)

                @pl.when(i > 0)
                def _():
                    pltpu.make_async_copy(
                        out_a, out.at[l0 - 2, kd, :, sub], osem_a).wait()

                _gather_chunk(row_v, idx_a, out_a)
                pltpu.async_copy(out_a, out.at[l0, kd, :, sub], osem_a)
                pltpu.make_async_copy(ids.at[pl.ds(l1 * B, B)],
                                      idx_b, isem_b).wait()

                @pl.when(i < HALF_L - 1)
                def _():
                    pltpu.async_copy(ids.at[pl.ds((l0 + 2) * B, B)],
                                     idx_a, isem_a)

                @pl.when(i > 0)
                def _():
                    pltpu.make_async_copy(
                        out_b, out.at[l1 - 2, kd, :, sub], osem_b).wait()

                _gather_chunk(row_v, idx_b, out_b)
                pltpu.async_copy(out_b, out.at[l1, kd, :, sub], osem_b)
                return c

            lax.fori_loop(0, HALF_L, li, 0)
            pltpu.make_async_copy(out_a, out.at[L - 2, kd, :, sub],
                                  osem_a).wait()
            pltpu.make_async_copy(out_b, out.at[L - 1, kd, :, sub],
                                  osem_b).wait()

    @pl.when(cid == 0)
    def _():
        do_table(ids_s, tab_s, out_s)

    @pl.when(cid == 1)
    def _():
        do_table(ids_t, tab_t, out_t)


_OUT5 = jax.ShapeDtypeStruct((L, D // 8, B // 128, 8, 128), jnp.float32)


@functools.partial(
    pl.kernel,
    out_type=(_OUT5, _OUT5),
    mesh=plsc.VectorSubcoreMesh(core_axis_name="c", subcore_axis_name="s"),
    compiler_params=pltpu.CompilerParams(
        use_tc_tiling_on_sc=False,
        needs_layout_passes=False,
    ),
    scratch_types=[
        pltpu.VMEM((V,), jnp.float32),
        pltpu.VMEM((B,), jnp.int32),
        pltpu.VMEM((B,), jnp.int32),
        pltpu.VMEM((B // 128, 128), jnp.float32),
        pltpu.VMEM((B // 128, 128), jnp.float32),
        pltpu.SemaphoreType.DMA,
        pltpu.SemaphoreType.DMA,
        pltpu.SemaphoreType.DMA,
        pltpu.SemaphoreType.DMA,
    ],
)
def _emb_kernel(ids_s, ids_t, tab_s, tab_t, out_s, out_t,
                row_v, idx_a, idx_b, out_a, out_b,
                isem_a, isem_b, osem_a, osem_b):
    _sc_body(ids_s, ids_t, tab_s, tab_t, out_s, out_t,
             row_v, idx_a, idx_b, out_a, out_b,
             isem_a, isem_b, osem_a, osem_b)


def kernel(src_ids, tgt_ids, src_table, tgt_table):
    ids_s = src_ids.T.reshape(-1).astype(jnp.int32)
    ids_t = tgt_ids.T.reshape(-1).astype(jnp.int32)
    o_s, o_t = _emb_kernel(ids_s, ids_t, src_table.T, tgt_table.T)

    def finish(o):
        return (o.transpose(0, 1, 3, 2, 4)
                 .reshape(L, D, B)
                 .transpose(2, 0, 1))

    return finish(o_s), finish(o_t)


# speedup vs baseline: 7.1006x; 1.0002x over previous
import functools

import jax
import jax.numpy as jnp
from jax import lax
from jax.experimental import pallas as pl
from jax.experimental.pallas import tpu as pltpu
from jax.experimental.pallas import tpu_sc as plsc

B, L, D = 4096, 50, 64
V = 100000
N = B * L
NC, NS = 2, 16
RPW = D // NS
HALF_L = L // 2


def _gather_chunk(row_v, idx_v, out_v):
    @plsc.parallel_loop(0, B // 16, 1, unroll=16)
    def _(b1):
        idx = idx_v[pl.ds(b1 * 16, 16)]
        vals = plsc.load_gather(row_v, [idx])
        out_v[b1 // 8, pl.ds((b1 % 8) * 16, 16)] = vals


def _sc_body(ids_s, ids_t, tab_s, tab_t, out_s, out_t,
             row_v, idx_a, idx_b, out_a, out_b,
             isem_a, isem_b, osem_a, osem_b):
    cid = lax.axis_index("c")
    sid = lax.axis_index("s")

    def do_table(ids, tab, out):
        d0 = sid * RPW
        for j in range(RPW):
            d = d0 + j
            kd = d // 8
            sub = d % 8
            pltpu.sync_copy(tab.at[d], row_v)
            pltpu.async_copy(ids.at[pl.ds(0, B)], idx_a, isem_a)

            def li(i, c):
                l0 = 2 * i
                l1 = 2 * i + 1
                pltpu.make_async_copy(ids.at[pl.ds(l0 * B, B)],
                                      idx_a, isem_a).wait()
                pltpu.async_copy(ids.at[pl.ds(l1 * B, B)], idx_b, isem_b)

                @pl.when(i > 0)
                def _():
                    pltpu.make_async_copy(
                        out_a, out.at[l0 - 2, kd, :, sub], osem_a).wait()

                _gather_chunk(row_v, idx_a, out_a)
                pltpu.async_copy(out_a, out.at[l0, kd, :, sub], osem_a)
                pltpu.make_async_copy(ids.at[pl.ds(l1 * B, B)],
                                      idx_b, isem_b).wait()

                @pl.when(i < HALF_L - 1)
                def _():
                    pltpu.async_copy(ids.at[pl.ds((l0 + 2) * B, B)],
                                     idx_a, isem_a)

                @pl.when(i > 0)
                def _():
                    pltpu.make_async_copy(
                        out_b, out.at[l1 - 2, kd, :, sub], osem_b).wait()

                _gather_chunk(row_v, idx_b, out_b)
                pltpu.async_copy(out_b, out.at[l1, kd, :, sub], osem_b)
                return c

            lax.fori_loop(0, HALF_L, li, 0)
            pltpu.make_async_copy(out_a, out.at[L - 2, kd, :, sub],
                                  osem_a).wait()
            pltpu.make_async_copy(out_b, out.at[L - 1, kd, :, sub],
                                  osem_b).wait()

    @pl.when(cid == 0)
    def _():
        do_table(ids_s, tab_s, out_s)

    @pl.when(cid == 1)
    def _():
        do_table(ids_t, tab_t, out_t)


_OUT5 = jax.ShapeDtypeStruct((L, D // 8, B // 128, 8, 128), jnp.float32)


@functools.partial(
    pl.kernel,
    out_type=(_OUT5, _OUT5),
    mesh=plsc.VectorSubcoreMesh(core_axis_name="c", subcore_axis_name="s"),
    compiler_params=pltpu.CompilerParams(
        use_tc_tiling_on_sc=False,
        needs_layout_passes=False,
    ),
    scratch_types=[
        pltpu.VMEM((V,), jnp.float32),
        pltpu.VMEM((B,), jnp.int32),
        pltpu.VMEM((B,), jnp.int32),
        pltpu.VMEM((B // 128, 128), jnp.float32),
        pltpu.VMEM((B // 128, 128), jnp.float32),
        pltpu.SemaphoreType.DMA,
        pltpu.SemaphoreType.DMA,
        pltpu.SemaphoreType.DMA,
        pltpu.SemaphoreType.DMA,
    ],
)
def _emb_kernel(ids_s, ids_t, tab_s, tab_t, out_s, out_t,
                row_v, idx_a, idx_b, out_a, out_b,
                isem_a, isem_b, osem_a, osem_b):
    _sc_body(ids_s, ids_t, tab_s, tab_t, out_s, out_t,
             row_v, idx_a, idx_b, out_a, out_b,
             isem_a, isem_b, osem_a, osem_b)


def kernel(src_ids, tgt_ids, src_table, tgt_table):
    ids_s = src_ids.T.reshape(-1).astype(jnp.int32)
    ids_t = tgt_ids.T.reshape(-1).astype(jnp.int32)
    o_s, o_t = _emb_kernel(ids_s, ids_t, src_table.T, tgt_table.T)

    def finish(o):
        return (o.transpose(0, 1, 3, 2, 4)
                 .reshape(L, D, B)
                 .transpose(2, 0, 1))

    return finish(o_s), finish(o_t)


# speedup vs baseline: 7.1339x; 1.0047x over previous
import functools

import jax
import jax.numpy as jnp
from jax import lax
from jax.experimental import pallas as pl
from jax.experimental.pallas import tpu as pltpu
from jax.experimental.pallas import tpu_sc as plsc

B, L, D = 4096, 50, 64
V = 100000
N = B * L
NC, NS = 2, 16
NW = NC * NS
RPW = D // NW
HALF_L = L // 2


def _gather_chunk(row_v, idx_v, out_v):
    @plsc.parallel_loop(0, B // 16, 1, unroll=8)
    def _(b1):
        idx = idx_v[pl.ds(b1 * 16, 16)]
        vals = plsc.load_gather(row_v, [idx])
        out_v[b1 // 8, pl.ds((b1 % 8) * 16, 16)] = vals


def _sc_body(ids, tab, out,
             row_v, idx_a, idx_b, out_a, out_b,
             isem_a, isem_b, osem_a, osem_b):
    cid = lax.axis_index("c")
    sid = lax.axis_index("s")
    wid = sid * NC + cid
    d0 = wid * RPW
    for j in range(RPW):
        d = d0 + j
        kd = d // 8
        sub = d % 8
        pltpu.sync_copy(tab.at[d], row_v)
        pltpu.async_copy(ids.at[pl.ds(0, B)], idx_a, isem_a)

        def li(i, c):
            l0 = 2 * i
            l1 = 2 * i + 1
            pltpu.make_async_copy(ids.at[pl.ds(l0 * B, B)],
                                  idx_a, isem_a).wait()
            pltpu.async_copy(ids.at[pl.ds(l1 * B, B)], idx_b, isem_b)

            @pl.when(i > 0)
            def _():
                pltpu.make_async_copy(
                    out_a, out.at[l0 - 2, kd, :, sub], osem_a).wait()

            _gather_chunk(row_v, idx_a, out_a)
            pltpu.async_copy(out_a, out.at[l0, kd, :, sub], osem_a)
            pltpu.make_async_copy(ids.at[pl.ds(l1 * B, B)],
                                  idx_b, isem_b).wait()

            @pl.when(i < HALF_L - 1)
            def _():
                pltpu.async_copy(ids.at[pl.ds((l0 + 2) * B, B)],
                                 idx_a, isem_a)

            @pl.when(i > 0)
            def _():
                pltpu.make_async_copy(
                    out_b, out.at[l1 - 2, kd, :, sub], osem_b).wait()

            _gather_chunk(row_v, idx_b, out_b)
            pltpu.async_copy(out_b, out.at[l1, kd, :, sub], osem_b)
            return c

        lax.fori_loop(0, HALF_L, li, 0)
        pltpu.make_async_copy(out_a, out.at[L - 2, kd, :, sub],
                              osem_a).wait()
        pltpu.make_async_copy(out_b, out.at[L - 1, kd, :, sub],
                              osem_b).wait()


_OUT5 = jax.ShapeDtypeStruct((L, D // 8, B // 128, 8, 128), jnp.float32)


@functools.partial(
    pl.kernel,
    out_type=_OUT5,
    mesh=plsc.VectorSubcoreMesh(core_axis_name="c", subcore_axis_name="s"),
    compiler_params=pltpu.CompilerParams(
        use_tc_tiling_on_sc=False,
        needs_layout_passes=False,
    ),
    scratch_types=[
        pltpu.VMEM((V,), jnp.float32),
        pltpu.VMEM((B,), jnp.int32),
        pltpu.VMEM((B,), jnp.int32),
        pltpu.VMEM((B // 128, 128), jnp.float32),
        pltpu.VMEM((B // 128, 128), jnp.float32),
        pltpu.SemaphoreType.DMA,
        pltpu.SemaphoreType.DMA,
        pltpu.SemaphoreType.DMA,
        pltpu.SemaphoreType.DMA,
    ],
)
def _emb_kernel(ids, tab, out,
                row_v, idx_a, idx_b, out_a, out_b,
                isem_a, isem_b, osem_a, osem_b):
    _sc_body(ids, tab, out,
             row_v, idx_a, idx_b, out_a, out_b,
             isem_a, isem_b, osem_a, osem_b)


def kernel(src_ids, tgt_ids, src_table, tgt_table):
    ids_s = src_ids.T.reshape(-1).astype(jnp.int32)
    ids_t = tgt_ids.T.reshape(-1).astype(jnp.int32)
    o_s = _emb_kernel(ids_s, src_table.T)
    o_t = _emb_kernel(ids_t, tgt_table.T)

    def finish(o):
        return (o.transpose(0, 1, 3, 2, 4)
                 .reshape(L, D, B)
                 .transpose(2, 0, 1))

    return finish(o_s), finish(o_t)


# speedup vs baseline: 13.4001x; 1.8784x over previous
import functools

import jax
import jax.numpy as jnp
from jax import lax
from jax.experimental import pallas as pl
from jax.experimental.pallas import tpu as pltpu
from jax.experimental.pallas import tpu_sc as plsc

B, L, D = 4096, 50, 64
V = 100000
N = B * L
NC, NS = 2, 16
NW = NC * NS
RPW = D // NW
HALF_L = L // 2


def _gather_chunk(row_v, idx_v, out_v):
    @plsc.parallel_loop(0, B // 16, 1, unroll=8)
    def _(b1):
        idx = idx_v[pl.ds(b1 * 16, 16)]
        vals = plsc.load_gather(row_v, [idx])
        out_v[b1 // 8, pl.ds((b1 % 8) * 16, 16)] = vals


def _sc_body(ids_hbm, tab, out,
             row_v, idx_a, idx_b, out_a, out_b, ids_sp,
             isem_a, isem_b, osem_a, osem_b):
    cid = lax.axis_index("c")
    sid = lax.axis_index("s")
    wid = sid * NC + cid
    @pl.when(sid == 0)
    def _():
        pltpu.sync_copy(ids_hbm, ids_sp)

    plsc.subcore_barrier()
    ids = ids_sp
    d0 = wid * RPW
    for j in range(RPW):
        d = d0 + j
        kd = d // 8
        sub = d % 8
        pltpu.sync_copy(tab.at[d], row_v)
        pltpu.async_copy(ids.at[pl.ds(0, B)], idx_a, isem_a)

        def li(i, c):
            l0 = 2 * i
            l1 = 2 * i + 1
            pltpu.make_async_copy(ids.at[pl.ds(l0 * B, B)],
                                  idx_a, isem_a).wait()
            pltpu.async_copy(ids.at[pl.ds(l1 * B, B)], idx_b, isem_b)

            @pl.when(i > 0)
            def _():
                pltpu.make_async_copy(
                    out_a, out.at[l0 - 2, kd, :, sub], osem_a).wait()

            _gather_chunk(row_v, idx_a, out_a)
            pltpu.async_copy(out_a, out.at[l0, kd, :, sub], osem_a)
            pltpu.make_async_copy(ids.at[pl.ds(l1 * B, B)],
                                  idx_b, isem_b).wait()

            @pl.when(i < HALF_L - 1)
            def _():
                pltpu.async_copy(ids.at[pl.ds((l0 + 2) * B, B)],
                                 idx_a, isem_a)

            @pl.when(i > 0)
            def _():
                pltpu.make_async_copy(
                    out_b, out.at[l1 - 2, kd, :, sub], osem_b).wait()

            _gather_chunk(row_v, idx_b, out_b)
            pltpu.async_copy(out_b, out.at[l1, kd, :, sub], osem_b)
            return c

        lax.fori_loop(0, HALF_L, li, 0)
        pltpu.make_async_copy(out_a, out.at[L - 2, kd, :, sub],
                              osem_a).wait()
        pltpu.make_async_copy(out_b, out.at[L - 1, kd, :, sub],
                              osem_b).wait()


_OUT5 = jax.ShapeDtypeStruct((L, D // 8, B // 128, 8, 128), jnp.float32)


@functools.partial(
    pl.kernel,
    out_type=_OUT5,
    mesh=plsc.VectorSubcoreMesh(core_axis_name="c", subcore_axis_name="s"),
    compiler_params=pltpu.CompilerParams(
        use_tc_tiling_on_sc=False,
        needs_layout_passes=False,
    ),
    scratch_types=[
        pltpu.VMEM((V,), jnp.float32),
        pltpu.VMEM((B,), jnp.int32),
        pltpu.VMEM((B,), jnp.int32),
        pltpu.VMEM((B // 128, 128), jnp.float32),
        pltpu.VMEM((B // 128, 128), jnp.float32),
        pltpu.VMEM_SHARED((N,), jnp.int32),
        pltpu.SemaphoreType.DMA,
        pltpu.SemaphoreType.DMA,
        pltpu.SemaphoreType.DMA,
        pltpu.SemaphoreType.DMA,
    ],
)
def _emb_kernel(ids, tab, out,
                row_v, idx_a, idx_b, out_a, out_b, ids_sp,
                isem_a, isem_b, osem_a, osem_b):
    _sc_body(ids, tab, out,
             row_v, idx_a, idx_b, out_a, out_b, ids_sp,
             isem_a, isem_b, osem_a, osem_b)


def kernel(src_ids, tgt_ids, src_table, tgt_table):
    ids_s = src_ids.T.reshape(-1).astype(jnp.int32)
    ids_t = tgt_ids.T.reshape(-1).astype(jnp.int32)
    o_s = _emb_kernel(ids_s, src_table.T)
    o_t = _emb_kernel(ids_t, tgt_table.T)

    def finish(o):
        return (o.transpose(0, 1, 3, 2, 4)
                 .reshape(L, D, B)
                 .transpose(2, 0, 1))

    return finish(o_s), finish(o_t)
